# baseline (speedup 1.0000x reference)
import jax
import jax.numpy as jnp
from jax.experimental import pallas as pl
from jax.experimental.pallas import tpu as pltpu

N = 10000
F = 128
BM1 = 400
BM2 = 400
HSCALE = 64.0


def _layer1_body(adj_ref, src_ref, srcr_ref, wlt_ref, wrt_ref, bias_ref,
                 h8_ref, adj8_ref):
    a = adj_ref[...]
    q32 = pltpu.bitcast(a.astype(jnp.float8_e4m3fn), jnp.int32)
    adj8_ref[...] = q32.reshape(1, BM1 // 4, N)
    acc = jnp.dot(a.astype(jnp.bfloat16), src_ref[...].astype(jnp.bfloat16),
                  preferred_element_type=jnp.float32)
    r = jnp.dot(acc.astype(jnp.bfloat16), wlt_ref[...].astype(jnp.bfloat16),
                preferred_element_type=jnp.float32)
    r += jnp.dot(srcr_ref[...].astype(jnp.bfloat16),
                 wrt_ref[...].astype(jnp.bfloat16),
                 preferred_element_type=jnp.float32)
    r += bias_ref[...]
    denom = jnp.maximum(jnp.sum(jnp.abs(r), axis=1, keepdims=True), 1e-12)
    r = jnp.maximum(r / denom, 0.0)
    h8_ref[...] = (r * HSCALE).astype(jnp.float8_e4m3fn)


def _layer2_body(adj8_ref, h8_ref, srcr_ref, wlt_ref, wrt_ref, bias_ref,
                 out_ref):
    a8 = pltpu.bitcast(adj8_ref[0], jnp.float8_e4m3fn)
    acc = jnp.dot(a8, h8_ref[...], preferred_element_type=jnp.float32)
    r = jnp.dot(acc.astype(jnp.bfloat16), wlt_ref[...].astype(jnp.bfloat16),
                preferred_element_type=jnp.float32)
    r += jnp.dot(srcr_ref[...].astype(jnp.bfloat16),
                 wrt_ref[...].astype(jnp.bfloat16),
                 preferred_element_type=jnp.float32)
    r += bias_ref[...]
    r = r - jnp.max(r, axis=1, keepdims=True)
    r = r - jnp.log(jnp.sum(jnp.exp(r), axis=1, keepdims=True))
    out_ref[...] = r


def _small_specs(bm):
    return [
        pl.BlockSpec((bm, F), lambda i: (i, 0)),
        pl.BlockSpec((F, F), lambda i: (0, 0)),
        pl.BlockSpec((F, F), lambda i: (0, 0)),
        pl.BlockSpec((1, F), lambda i: (0, 0)),
    ]


def kernel(x, block, W_l1, b_l1, W_r1, b_r1, W_l2, b_l2, W_r2, b_r2):
    adj = block[0]
    b1 = (b_l1 + b_r1).reshape(1, F)
    b2 = (b_l2 + b_r2).reshape(1, F)

    h8, adj8 = pl.pallas_call(
        _layer1_body,
        grid=(N // BM1,),
        in_specs=[
            pl.BlockSpec((BM1, N), lambda i: (i, 0)),
            pl.BlockSpec((N, F), lambda i: (0, 0)),
        ] + _small_specs(BM1),
        out_specs=[
            pl.BlockSpec((BM1, F), lambda i: (i, 0)),
            pl.BlockSpec((1, BM1 // 4, N), lambda i: (i, 0, 0)),
        ],
        out_shape=[
            jax.ShapeDtypeStruct((N, F), jnp.float8_e4m3fn),
            jax.ShapeDtypeStruct((N // BM1, BM1 // 4, N), jnp.int32),
        ],
    )(adj, x, x, W_l1.T, W_r1.T, b1)

    return pl.pallas_call(
        _layer2_body,
        grid=(N // BM2,),
        in_specs=[
            pl.BlockSpec((1, BM2 // 4, N), lambda i: (i, 0, 0)),
            pl.BlockSpec((N, F), lambda i: (0, 0)),
        ] + _small_specs(BM2),
        out_specs=pl.BlockSpec((BM2, F), lambda i: (i, 0)),
        out_shape=jax.ShapeDtypeStruct((N, F), jnp.float32),
    )(adj8, h8, h8, W_l2.T / HSCALE, W_r2.T / HSCALE, b2)


# speedup vs baseline: 1.0057x; 1.0057x over previous
import jax
import jax.numpy as jnp
from jax.experimental import pallas as pl

N = 10000
F = 128
BM1 = 400
BM2 = 400
HSCALE = 64.0


def _layer1_body(adj_ref, src_ref, srcr_ref, wlt_ref, wrt_ref, bias_ref,
                 h8_ref, adj8_ref):
    a = adj_ref[...]
    adj8_ref[...] = a.astype(jnp.float8_e4m3fn)
    acc = jnp.dot(a.astype(jnp.bfloat16), src_ref[...].astype(jnp.bfloat16),
                  preferred_element_type=jnp.float32)
    r = jnp.dot(acc.astype(jnp.bfloat16), wlt_ref[...].astype(jnp.bfloat16),
                preferred_element_type=jnp.float32)
    r += jnp.dot(srcr_ref[...].astype(jnp.bfloat16),
                 wrt_ref[...].astype(jnp.bfloat16),
                 preferred_element_type=jnp.float32)
    r += bias_ref[...]
    denom = jnp.maximum(jnp.sum(jnp.abs(r), axis=1, keepdims=True), 1e-12)
    r = jnp.maximum(r / denom, 0.0)
    h8_ref[...] = (r * HSCALE).astype(jnp.float8_e4m3fn)


def _layer2_body(adj8_ref, h8_ref, srcr_ref, wlt_ref, wrt_ref, bias_ref,
                 out_ref):
    acc = jnp.dot(adj8_ref[...], h8_ref[...],
                  preferred_element_type=jnp.float32)
    r = jnp.dot(acc.astype(jnp.bfloat16), wlt_ref[...].astype(jnp.bfloat16),
                preferred_element_type=jnp.float32)
    r += jnp.dot(srcr_ref[...].astype(jnp.bfloat16),
                 wrt_ref[...].astype(jnp.bfloat16),
                 preferred_element_type=jnp.float32)
    r += bias_ref[...]
    r = r - jnp.max(r, axis=1, keepdims=True)
    r = r - jnp.log(jnp.sum(jnp.exp(r), axis=1, keepdims=True))
    out_ref[...] = r


def _small_specs(bm):
    return [
        pl.BlockSpec((bm, F), lambda i: (i, 0)),
        pl.BlockSpec((F, F), lambda i: (0, 0)),
        pl.BlockSpec((F, F), lambda i: (0, 0)),
        pl.BlockSpec((1, F), lambda i: (0, 0)),
    ]


def kernel(x, block, W_l1, b_l1, W_r1, b_r1, W_l2, b_l2, W_r2, b_r2):
    adj = block[0]
    b1 = (b_l1 + b_r1).reshape(1, F)
    b2 = (b_l2 + b_r2).reshape(1, F)

    h8, adj8 = pl.pallas_call(
        _layer1_body,
        grid=(N // BM1,),
        in_specs=[
            pl.BlockSpec((BM1, N), lambda i: (i, 0)),
            pl.BlockSpec((N, F), lambda i: (0, 0)),
        ] + _small_specs(BM1),
        out_specs=[
            pl.BlockSpec((BM1, F), lambda i: (i, 0)),
            pl.BlockSpec((BM1, N), lambda i: (i, 0)),
        ],
        out_shape=[
            jax.ShapeDtypeStruct((N, F), jnp.float8_e4m3fn),
            jax.ShapeDtypeStruct((N, N), jnp.float8_e4m3fn),
        ],
    )(adj, x, x, W_l1.T, W_r1.T, b1)

    return pl.pallas_call(
        _layer2_body,
        grid=(N // BM2,),
        in_specs=[
            pl.BlockSpec((BM2, N), lambda i: (i, 0)),
            pl.BlockSpec((N, F), lambda i: (0, 0)),
        ] + _small_specs(BM2),
        out_specs=pl.BlockSpec((BM2, F), lambda i: (i, 0)),
        out_shape=jax.ShapeDtypeStruct((N, F), jnp.float32),
    )(adj8, h8, h8, W_l2.T / HSCALE, W_r2.T / HSCALE, b2)
